# Initial kernel scaffold; baseline (speedup 1.0000x reference)
import jax
import jax.numpy as jnp
from jax.experimental import pallas as pl


def kernel(input_feat, table):
    raise NotImplementedError("write your pallas kernel here")


# speedup vs baseline: 1.8507x; 1.8507x over previous
import functools

import jax
import jax.numpy as jnp
import numpy as np
from jax import lax
from jax.experimental import pallas as pl
from jax.experimental.pallas import tpu as pltpu
from jax.experimental.pallas import tpu_sc as plsc

_FEAT_CNT = [40000] * 26
_F = len(_FEAT_CNT)
_D = 32
_B = 16384
_L = 16

_INFO = plsc.get_sparse_core_info()
_NC, _NS = _INFO.num_cores, _INFO.num_subcores
_NW = _NC * _NS
_PER_W = _B // _NW
_CHUNK_E = 64
_NCHUNK = _PER_W // _CHUNK_E
_ROWS = _CHUNK_E * _F
_IDX_W = 128
_IDX_H = _ROWS // _IDX_W
_NVREG = _ROWS // _L
_PAT_N = 13 * _L

_mesh = plsc.VectorSubcoreMesh(core_axis_name="c", subcore_axis_name="s")


@functools.partial(
    pl.kernel,
    out_type=jax.ShapeDtypeStruct((_B, _D), jnp.float32),
    mesh=_mesh,
    scratch_types=[
        pltpu.VMEM((_ROWS,), jnp.int32),
        pltpu.VMEM((_PAT_N,), jnp.int32),
        pltpu.VMEM((_IDX_H, _IDX_W), jnp.int32),
        pltpu.VMEM((_ROWS, _D), jnp.float32),
        pltpu.VMEM((_CHUNK_E, _D), jnp.float32),
        pltpu.SemaphoreType.DMA,
    ],
    compiler_params=pltpu.CompilerParams(use_tc_tiling_on_sc=False),
)
def _embed_pool(feat_hbm, pat_hbm, table_hbm, out_hbm,
                feat_v, pat_v, idx_v, rows_v, out_v, sem):
    wid = lax.axis_index("s") * _NC + lax.axis_index("c")
    pltpu.sync_copy(pat_hbm, pat_v)

    @pl.loop(0, _NCHUNK)
    def _chunk(c):
        e_base = wid * _PER_W + c * _CHUNK_E
        pltpu.sync_copy(feat_hbm.at[pl.ds(e_base * _F, _ROWS)], feat_v)

        for i in range(_NVREG):
            f = feat_v[pl.ds(i * _L, _L)]
            p = pat_v[pl.ds((i % 13) * _L, _L)]
            idx_v[i // (_IDX_W // _L), pl.ds((i % (_IDX_W // _L)) * _L, _L)] = f + p

        copies = []
        for j in range(_IDX_H):
            copies.append(
                pltpu.async_copy(
                    table_hbm.at[idx_v.at[j]],
                    rows_v.at[pl.ds(j * _IDX_W, _IDX_W)],
                    sem,
                )
            )
        for cp in copies:
            cp.wait()

        @pl.loop(0, _CHUNK_E)
        def _elem(e):
            r0 = e * _F
            acc0 = rows_v[r0, pl.ds(0, _L)]
            acc1 = rows_v[r0, pl.ds(_L, _L)]
            for f in range(1, _F):
                acc0 += rows_v[r0 + f, pl.ds(0, _L)]
                acc1 += rows_v[r0 + f, pl.ds(_L, _L)]
            scale = jnp.float32(1.0 / _F)
            out_v[e, pl.ds(0, _L)] = jnp.maximum(acc0 * scale, 0.0)
            out_v[e, pl.ds(_L, _L)] = jnp.maximum(acc1 * scale, 0.0)

        pltpu.sync_copy(out_v, out_hbm.at[pl.ds(e_base, _CHUNK_E)])


def kernel(input_feat, table):
    acu = np.concatenate([[0], np.cumsum(_FEAT_CNT)[:-1]]).astype(np.int32)
    pattern = jnp.asarray(acu[np.arange(_PAT_N) % _F], dtype=jnp.int32)
    feat_flat = input_feat.reshape(-1)
    return _embed_pool(feat_flat, pattern, table)


# speedup vs baseline: 1.8757x; 1.0135x over previous
import functools

import jax
import jax.numpy as jnp
from jax import lax
from jax.experimental import pallas as pl
from jax.experimental.pallas import tpu as pltpu
from jax.experimental.pallas import tpu_sc as plsc

_FEAT_CNT = [40000] * 26
_F = len(_FEAT_CNT)
_D = 32
_B = 16384
_L = 16

_INFO = plsc.get_sparse_core_info()
_NC, _NS = _INFO.num_cores, _INFO.num_subcores
_NW = _NC * _NS
_PER_W = _B // _NW
_CHUNK_E = 64
_NCHUNK = _PER_W // _CHUNK_E
_ROWS = _CHUNK_E * _F
_IDX_W = 128
_IDX_H = _ROWS // _IDX_W

_ACU = [sum(_FEAT_CNT[:f]) for f in range(_F)]

_mesh = plsc.VectorSubcoreMesh(core_axis_name="c", subcore_axis_name="s")


@functools.partial(
    pl.kernel,
    out_type=jax.ShapeDtypeStruct((_B, _D), jnp.float32),
    mesh=_mesh,
    scratch_types=[
        pltpu.VMEM((_F, _CHUNK_E), jnp.int32),
        pltpu.VMEM((_IDX_H, _IDX_W), jnp.int32),
        pltpu.VMEM((_ROWS, _D), jnp.float32),
        pltpu.VMEM((_CHUNK_E, _D), jnp.float32),
        pltpu.SemaphoreType.DMA,
    ],
    compiler_params=pltpu.CompilerParams(use_tc_tiling_on_sc=False),
)
def _embed_pool(featT_hbm, table_hbm, out_hbm,
                feat_v, idx_v, rows_v, out_v, sem):
    wid = lax.axis_index("s") * _NC + lax.axis_index("c")

    @pl.loop(0, _NCHUNK)
    def _chunk(c):
        e_base = wid * _PER_W + c * _CHUNK_E
        pltpu.sync_copy(featT_hbm.at[:, pl.ds(e_base, _CHUNK_E)], feat_v)

        for f in range(_F):
            off = jnp.int32(_ACU[f])
            for k in range(_CHUNK_E // _L):
                p = f * _CHUNK_E + k * _L
                idx_v[p // _IDX_W, pl.ds(p % _IDX_W, _L)] = (
                    feat_v[f, pl.ds(k * _L, _L)] + off
                )

        copies = []
        for j in range(_IDX_H):
            copies.append(
                pltpu.async_copy(
                    table_hbm.at[idx_v.at[j]],
                    rows_v.at[pl.ds(j * _IDX_W, _IDX_W)],
                    sem,
                )
            )
        for cp in copies:
            cp.wait()

        @pl.loop(0, _CHUNK_E)
        def _elem(e):
            acc0 = rows_v[e, pl.ds(0, _L)]
            acc1 = rows_v[e, pl.ds(_L, _L)]
            for f in range(1, _F):
                acc0 += rows_v[f * _CHUNK_E + e, pl.ds(0, _L)]
                acc1 += rows_v[f * _CHUNK_E + e, pl.ds(_L, _L)]
            scale = jnp.float32(1.0 / _F)
            out_v[e, pl.ds(0, _L)] = jnp.maximum(acc0 * scale, 0.0)
            out_v[e, pl.ds(_L, _L)] = jnp.maximum(acc1 * scale, 0.0)

        pltpu.sync_copy(out_v, out_hbm.at[pl.ds(e_base, _CHUNK_E)])


def kernel(input_feat, table):
    return _embed_pool(input_feat.T, table)


# speedup vs baseline: 2.2774x; 1.2142x over previous
import functools

import jax
import jax.numpy as jnp
from jax import lax
from jax.experimental import pallas as pl
from jax.experimental.pallas import tpu as pltpu
from jax.experimental.pallas import tpu_sc as plsc

_FEAT_CNT = [40000] * 26
_F = len(_FEAT_CNT)
_D = 32
_B = 16384
_L = 16

_INFO = plsc.get_sparse_core_info()
_NC, _NS = _INFO.num_cores, _INFO.num_subcores
_NW = _NC * _NS
_PER_W = _B // _NW
_CHUNK_E = 64
_NCHUNK = _PER_W // _CHUNK_E
_ROWS = _CHUNK_E * _F
_IDX_W = 128
_IDX_H = _ROWS // _IDX_W

_ACU = [sum(_FEAT_CNT[:f]) for f in range(_F)]

_mesh = plsc.VectorSubcoreMesh(core_axis_name="c", subcore_axis_name="s")


@functools.partial(
    pl.kernel,
    out_type=jax.ShapeDtypeStruct((_B, _D), jnp.float32),
    mesh=_mesh,
    scratch_types=[
        pltpu.VMEM((_F, _CHUNK_E), jnp.int32),
        pltpu.VMEM((_IDX_H, _IDX_W), jnp.int32),
        pltpu.VMEM((_ROWS, _D), jnp.float32),
        pltpu.VMEM((_CHUNK_E, _D), jnp.float32),
        pltpu.SemaphoreType.DMA,
    ],
    compiler_params=pltpu.CompilerParams(use_tc_tiling_on_sc=False),
)
def _embed_pool(featT_hbm, table_hbm, out_hbm,
                feat_v, idx_v, rows_v, out_v, sem):
    wid = lax.axis_index("s") * _NC + lax.axis_index("c")

    @pl.loop(0, _NCHUNK)
    def _chunk(c):
        e_base = wid * _PER_W + c * _CHUNK_E
        pltpu.sync_copy(featT_hbm.at[:, pl.ds(e_base, _CHUNK_E)], feat_v)

        for f in range(_F):
            off = jnp.int32(_ACU[f])
            for k in range(_CHUNK_E // _L):
                p = f * _CHUNK_E + k * _L
                idx_v[p // _IDX_W, pl.ds(p % _IDX_W, _L)] = (
                    feat_v[f, pl.ds(k * _L, _L)] + off
                )

        copies = []
        for j in range(_IDX_H):
            copies.append(
                pltpu.async_copy(
                    table_hbm.at[idx_v.at[j]],
                    rows_v.at[pl.ds(j * _IDX_W, _IDX_W)],
                    sem,
                )
            )
        for cp in copies:
            cp.wait()

        @pl.loop(0, _CHUNK_E)
        def _elem(e):
            acc0 = rows_v[e, pl.ds(0, _L)]
            acc1 = rows_v[e, pl.ds(_L, _L)]
            for f in range(1, _F):
                acc0 += rows_v[f * _CHUNK_E + e, pl.ds(0, _L)]
                acc1 += rows_v[f * _CHUNK_E + e, pl.ds(_L, _L)]
            scale = jnp.float32(1.0 / _F)
            out_v[e, pl.ds(0, _L)] = jnp.maximum(acc0 * scale, 0.0)
            out_v[e, pl.ds(_L, _L)] = jnp.maximum(acc1 * scale, 0.0)

        pltpu.sync_copy(out_v, out_hbm.at[pl.ds(e_base, _CHUNK_E)])


_TCOLS = 3200
_TGRID = (_B and (1040000 // _TCOLS))


def _detile_body(w_ref, x_ref):
    y = w_ref[...].T.reshape(_TCOLS // 4, 4, _D)
    for m in range(4):
        x_ref[:, 32 * m:32 * (m + 1)] = y[:, m, :]


def _detile(tableT):
    return pl.pallas_call(
        _detile_body,
        out_shape=jax.ShapeDtypeStruct((1040000 // 4, 128), jnp.float32),
        grid=(_TGRID,),
        in_specs=[pl.BlockSpec((_D, _TCOLS), lambda i: (0, i))],
        out_specs=pl.BlockSpec((_TCOLS // 4, 128), lambda i: (i, 0)),
    )(tableT)


def kernel(input_feat, table):
    table_lin = _detile(table.T).reshape(1040000, _D)
    return _embed_pool(input_feat.T, table_lin)
